# baseline (speedup 1.0000x reference)
import jax
import jax.numpy as jnp
from jax import lax
from jax.experimental import pallas as pl
from jax.experimental.pallas import tpu as pltpu
from jax.experimental.pallas import tpu_sc as plsc

SEQ_LEN = 131072
NUM_ANCHORS = 5
PRE_N = 6000
POST_N = 300
NMS_THRESH = 0.7
NPAD = 6016
OUT_R = 512
NSC = 655360
NTILES = 32
CHUNK = NSC // NTILES
ROWS16 = NSC // 16
TROWS = CHUNK // 16
BASE_ANCHORS = jnp.array([[-4.0, 3.0], [-8.0, 7.0], [-16.0, 15.0], [-32.0, 31.0], [-64.0, 63.0]], dtype=jnp.float32)

ENC_SPEC = [(14, 32, 3, 1, 1, 16), (32, 16, 3, 1, 1, 8), (16, 8, 3, 2, 2, 4), (8, 4, 3, 2, 2, 2), (4, 2, 3, 3, 3, 1)]
DEC_SPEC = [(2, 4, 3, 3, 3, 2), (8, 8, 3, 2, 2, 4), (16, 16, 3, 2, 2, 8), (32, 32, 3, 1, 1, 16), (64, 32, 3, 1, 1, 16)]


def _conv1d(x, w, b=None, pad=0, dil=1, groups=1):
    y = jax.lax.conv_general_dilated(x, w, window_strides=(1,), padding=[(pad, pad)], rhs_dilation=(dil,), dimension_numbers=('NCH', 'OIH', 'NCH'), feature_group_count=groups)
    if b is not None:
        y = y + b[None, :, None]
    return y


def _batchnorm(x, g, b, eps=1e-5):
    m = x.mean(axis=(0, 2), keepdims=True)
    v = ((x - m) ** 2).mean(axis=(0, 2), keepdims=True)
    return g[None, :, None] * (x - m) / jnp.sqrt(v + eps) + b[None, :, None]


def _ads_conv(x, p, pad, dil):
    C = x.shape[1]
    h = _conv1d(x, p['dw_w'], p['dw_b'], pad=pad, dil=dil, groups=C)
    h = jax.nn.relu(h)
    ak = p['attn_w'].shape[-1]
    a = _conv1d(h, p['attn_w'], p['attn_b'], pad=(ak - 1) // 2, dil=1, groups=C)
    h = h * jax.nn.sigmoid(a)
    s = h.mean(axis=2)
    s = jax.nn.relu(s @ p['se_w1'].T + p['se_b1'])
    s = jax.nn.sigmoid(s @ p['se_w2'].T + p['se_b2'])
    h = h * s[:, :, None]
    return _conv1d(h, p['pw_w'], p['pw_b'])


def _backbone(sequence, params):
    L = sequence.shape[-1]
    out = sequence
    inter = []
    for p, (cin, cout, k, pad, dil, rr) in zip(params['enc'], ENC_SPEC):
        out = _batchnorm(jax.nn.relu(_ads_conv(out, p, pad, dil)), p['bn_g'], p['bn_b'])
        inter.append(out)
    inter.pop()
    for p, (cin, cout, k, pad, dil, rr) in zip(params['dec'][:-1], DEC_SPEC[:-1]):
        out = _batchnorm(jax.nn.relu(_ads_conv(out, p, pad, dil)), p['bn_g'], p['bn_b'])
        out = jnp.concatenate([out, inter.pop()], axis=1)
    p = params['dec'][-1]
    cin, cout, k, pad, dil, rr = DEC_SPEC[-1]
    feat = _batchnorm(jax.nn.relu(_ads_conv(out, p, pad, dil)), p['bn_g'], p['bn_b'])

    rp = params['rpn']
    r = _conv1d(feat, rp['dw_w'], rp['dw_b'], pad=1, dil=1, groups=32)
    r = _conv1d(r, rp['pw_w'], rp['pw_b'])
    r = _batchnorm(jax.nn.relu(r), rp['bn_g'], rp['bn_b'])

    cls = _conv1d(r, params['cls_w'], params['cls_b'])
    prob = jax.nn.sigmoid(cls).transpose(0, 2, 1)
    box = _conv1d(r, params['box_w'], params['box_b']).transpose(0, 2, 1)

    scores = prob.reshape(-1)
    deltas = box.reshape(-1, 2)
    shifts = jnp.arange(L, dtype=jnp.float32)
    anc = (shifts[:, None, None] + BASE_ANCHORS[None, :, :]).reshape(-1, 2)
    w = anc[:, 1] - anc[:, 0] + 1.0
    ctr = anc[:, 0] + 0.5 * w
    pred_ctr = deltas[:, 0] * w + ctr
    pred_w = jnp.exp(jnp.clip(deltas[:, 1], -10.0, 10.0)) * w
    s = jnp.clip(pred_ctr - 0.5 * pred_w, 0.0, L - 1.0)
    e = jnp.clip(pred_ctr + 0.5 * pred_w, 0.0, L - 1.0)
    return scores, s, e


_NBLK = NSC // 128 // 128


def _thresh_body(sc_ref, out_ref):
    def count_ge(b):
        def blk(k, acc):
            x = sc_ref[pl.ds(k * 128, 128), :]
            xb = jax.lax.bitcast_convert_type(x, jnp.int32)
            return acc + jnp.sum((xb >= b).astype(jnp.int32))
        return lax.fori_loop(0, _NBLK, blk, jnp.int32(0))

    def bs(it, lohi):
        lo, hi = lohi
        mid = (lo + hi) // 2
        big = count_ge(mid) >= PRE_N
        return jnp.where(big, mid, lo), jnp.where(big, hi, mid)

    lo, hi = lax.fori_loop(0, 31, bs, (jnp.int32(0), jnp.int32(0x3F800001)))
    tb = lo
    b6 = PRE_N - count_ge(tb + 1)

    rowi = jax.lax.broadcasted_iota(jnp.int32, (128, 128), 0)
    lanei = jax.lax.broadcasted_iota(jnp.int32, (128, 128), 1)

    def count_eq_le(K):
        def blk(k, acc):
            x = sc_ref[pl.ds(k * 128, 128), :]
            xb = jax.lax.bitcast_convert_type(x, jnp.int32)
            gidx = (k * 128 + rowi) * 128 + lanei
            return acc + jnp.sum(((xb == tb) & (gidx <= K)).astype(jnp.int32))
        return lax.fori_loop(0, _NBLK, blk, jnp.int32(0))

    def bs2(it, lohi):
        lo, hi = lohi
        mid = (lo + hi) // 2
        ok = count_eq_le(mid) >= b6
        return jnp.where(ok, lo, mid), jnp.where(ok, hi, mid)

    lo2, hi2 = lax.fori_loop(0, 20, bs2, (jnp.int32(-1), jnp.int32(NSC - 1)))
    lane = jax.lax.broadcasted_iota(jnp.int32, (1, 128), 1)
    out_ref[...] = jnp.where(lane == 0, tb, jnp.where(lane == 1, hi2, 0))


def _threshold(scores):
    bk = pl.pallas_call(
        _thresh_body,
        out_shape=jax.ShapeDtypeStruct((1, 128), jnp.int32),
    )(scores.reshape(NSC // 128, 128))
    t = jax.lax.bitcast_convert_type(bk[0, 0], jnp.float32)
    return t, bk[0, 1]


def _compact_sc_body(score_hbm, t_hbm, k_hbm, outidx_hbm, counts_hbm, sc_vm, idx_vm, t_vm, k_vm, cnt_vm):
    wid = lax.axis_index("s") * 2 + lax.axis_index("c")
    base = wid * CHUNK
    pltpu.sync_copy(score_hbm.at[pl.ds(base, CHUNK)], sc_vm)
    pltpu.sync_copy(t_hbm, t_vm)
    pltpu.sync_copy(k_hbm, k_vm)
    t = t_vm[...]
    kv = k_vm[...]
    iota16 = lax.iota(jnp.int32, 16)

    def step(j, cursor):
        lidx = j * 16 + iota16
        v = plsc.load_gather(sc_vm, [lidx])
        gidx = base + lidx
        m = (v > t) | ((v == t) & (gidx <= kv))
        pos = cursor + plsc.cumsum(m.astype(jnp.int32)) - 1
        plsc.store_scatter(idx_vm, [pos], gidx, mask=m)
        return cursor + jnp.sum(m.astype(jnp.int32))

    cursor = lax.fori_loop(0, TROWS, step, jnp.int32(0))
    cnt_vm[...] = jnp.full((16,), cursor, jnp.int32)
    pltpu.sync_copy(cnt_vm, counts_hbm.at[wid])
    pltpu.sync_copy(idx_vm, outidx_hbm.at[wid])


def _compact(scores, t, K):
    t16 = jnp.full((16,), t, jnp.float32)
    k16 = jnp.full((16,), K, jnp.int32)
    mesh = plsc.VectorSubcoreMesh(core_axis_name="c", subcore_axis_name="s")
    kfn = pl.kernel(
        _compact_sc_body,
        out_type=[
            jax.ShapeDtypeStruct((NTILES, NPAD), jnp.int32),
            jax.ShapeDtypeStruct((NTILES, 16), jnp.int32),
        ],
        mesh=mesh,
        compiler_params=pltpu.CompilerParams(needs_layout_passes=False),
        scratch_types=[
            pltpu.VMEM((CHUNK,), jnp.float32),
            pltpu.VMEM((NPAD,), jnp.int32),
            pltpu.VMEM((16,), jnp.float32),
            pltpu.VMEM((16,), jnp.int32),
            pltpu.VMEM((16,), jnp.int32),
        ],
    )
    outidx, counts = kfn(scores, t16, k16)
    return outidx, counts[:, 0]


def _nms_sel_body(scr, ssr, eer, scc, ssc, eec, osc, oss, oee, live_ref):
    osc[...] = jnp.zeros_like(osc)
    oss[...] = jnp.zeros_like(oss)
    oee[...] = jnp.zeros_like(oee)
    live_ref[...] = scr[...]
    s_row = ssr[0:1, :]
    e_row = eer[0:1, :]
    lens = e_row - s_row + 1.0
    lane = jax.lax.broadcasted_iota(jnp.int32, (1, NPAD), 1)

    def cond_fn(st):
        cursor, done = st
        return jnp.logical_and(cursor < POST_N, done == 0)

    def body_fn(st):
        cursor, done = st
        live = live_ref[0:1, :]
        mx = jnp.max(live)
        is_done = mx < -0.5

        @pl.when(jnp.logical_not(is_done))
        def _():
            eq = live == mx
            p = jnp.min(jnp.where(eq, lane, jnp.int32(NPAD)))
            si = ssc[pl.ds(p, 1), :]
            ei = eec[pl.ds(p, 1), :]
            li = ei - si + 1.0
            inter = jnp.maximum(0.0, jnp.minimum(ei, e_row) - jnp.maximum(si, s_row) + 1.0)
            iou = inter / (li + lens - inter)
            live_ref[0:1, :] = jnp.where(iou > NMS_THRESH, -1.0, live)
            osc[pl.ds(cursor, 1), :] = scc[pl.ds(p, 1), :]
            oss[pl.ds(cursor, 1), :] = si
            oee[pl.ds(cursor, 1), :] = ei

        return cursor + jnp.where(is_done, 0, 1).astype(jnp.int32), jnp.where(is_done, 1, 0).astype(jnp.int32)

    lax.while_loop(cond_fn, body_fn, (jnp.int32(0), jnp.int32(0)))


def _nms_topk(sc, ss, ee):
    pad = NPAD - PRE_N
    neg = jnp.full((pad,), -3.0e9, jnp.float32)
    scr = jnp.concatenate([sc, jnp.full((pad,), -1.0, jnp.float32)]).reshape(1, NPAD)
    ssr = jnp.concatenate([ss, neg]).reshape(1, NPAD)
    eer = jnp.concatenate([ee, neg]).reshape(1, NPAD)
    scc = scr.reshape(NPAD, 1)
    ssc = ssr.reshape(NPAD, 1)
    eec = eer.reshape(NPAD, 1)
    out = pl.pallas_call(
        _nms_sel_body,
        out_shape=[jax.ShapeDtypeStruct((OUT_R, 1), jnp.float32)] * 3,
        scratch_shapes=[pltpu.VMEM((1, NPAD), jnp.float32)],
    )(scr, ssr, eer, scc, ssc, eec)
    osc, oss, oee = out
    return jnp.stack([osc[:POST_N, 0], oss[:POST_N, 0], oee[:POST_N, 0]], axis=-1)


def kernel(sequence, params):
    scores, s, e = _backbone(sequence, params)
    t, K = _threshold(scores)
    outidx, cnt = _compact(scores, t, K)
    offs0 = jnp.concatenate([jnp.zeros((1,), jnp.int32), jnp.cumsum(cnt)])
    p = jnp.arange(PRE_N, dtype=jnp.int32)
    tile = jnp.searchsorted(offs0, p, side='right').astype(jnp.int32) - 1
    src = tile * NPAD + (p - offs0[tile])
    fidx = outidx.reshape(-1)[src]
    out3 = _nms_topk(scores[fidx], s[fidx], e[fidx])
    return out3[:, None, :]


# speedup vs baseline: 1.0284x; 1.0284x over previous
import jax
import jax.numpy as jnp
from jax import lax
from jax.experimental import pallas as pl
from jax.experimental.pallas import tpu as pltpu
from jax.experimental.pallas import tpu_sc as plsc

SEQ_LEN = 131072
NUM_ANCHORS = 5
PRE_N = 6000
POST_N = 300
NMS_THRESH = 0.7
NPAD = 6016
OUT_R = 512
NSC = 655360
NTILES = 32
CHUNK = NSC // NTILES
ROWS16 = NSC // 16
TROWS = CHUNK // 16
BASE_ANCHORS = jnp.array([[-4.0, 3.0], [-8.0, 7.0], [-16.0, 15.0], [-32.0, 31.0], [-64.0, 63.0]], dtype=jnp.float32)

ENC_SPEC = [(14, 32, 3, 1, 1, 16), (32, 16, 3, 1, 1, 8), (16, 8, 3, 2, 2, 4), (8, 4, 3, 2, 2, 2), (4, 2, 3, 3, 3, 1)]
DEC_SPEC = [(2, 4, 3, 3, 3, 2), (8, 8, 3, 2, 2, 4), (16, 16, 3, 2, 2, 8), (32, 32, 3, 1, 1, 16), (64, 32, 3, 1, 1, 16)]


def _conv1d(x, w, b=None, pad=0, dil=1, groups=1):
    y = jax.lax.conv_general_dilated(x, w, window_strides=(1,), padding=[(pad, pad)], rhs_dilation=(dil,), dimension_numbers=('NCH', 'OIH', 'NCH'), feature_group_count=groups)
    if b is not None:
        y = y + b[None, :, None]
    return y


def _batchnorm(x, g, b, eps=1e-5):
    m = x.mean(axis=(0, 2), keepdims=True)
    v = ((x - m) ** 2).mean(axis=(0, 2), keepdims=True)
    return g[None, :, None] * (x - m) / jnp.sqrt(v + eps) + b[None, :, None]


def _ads_conv(x, p, pad, dil):
    C = x.shape[1]
    h = _conv1d(x, p['dw_w'], p['dw_b'], pad=pad, dil=dil, groups=C)
    h = jax.nn.relu(h)
    ak = p['attn_w'].shape[-1]
    a = _conv1d(h, p['attn_w'], p['attn_b'], pad=(ak - 1) // 2, dil=1, groups=C)
    h = h * jax.nn.sigmoid(a)
    s = h.mean(axis=2)
    s = jax.nn.relu(s @ p['se_w1'].T + p['se_b1'])
    s = jax.nn.sigmoid(s @ p['se_w2'].T + p['se_b2'])
    h = h * s[:, :, None]
    return _conv1d(h, p['pw_w'], p['pw_b'])


def _backbone(sequence, params):
    L = sequence.shape[-1]
    out = sequence
    inter = []
    for p, (cin, cout, k, pad, dil, rr) in zip(params['enc'], ENC_SPEC):
        out = _batchnorm(jax.nn.relu(_ads_conv(out, p, pad, dil)), p['bn_g'], p['bn_b'])
        inter.append(out)
    inter.pop()
    for p, (cin, cout, k, pad, dil, rr) in zip(params['dec'][:-1], DEC_SPEC[:-1]):
        out = _batchnorm(jax.nn.relu(_ads_conv(out, p, pad, dil)), p['bn_g'], p['bn_b'])
        out = jnp.concatenate([out, inter.pop()], axis=1)
    p = params['dec'][-1]
    cin, cout, k, pad, dil, rr = DEC_SPEC[-1]
    feat = _batchnorm(jax.nn.relu(_ads_conv(out, p, pad, dil)), p['bn_g'], p['bn_b'])

    rp = params['rpn']
    r = _conv1d(feat, rp['dw_w'], rp['dw_b'], pad=1, dil=1, groups=32)
    r = _conv1d(r, rp['pw_w'], rp['pw_b'])
    r = _batchnorm(jax.nn.relu(r), rp['bn_g'], rp['bn_b'])

    cls = _conv1d(r, params['cls_w'], params['cls_b'])
    prob = jax.nn.sigmoid(cls).transpose(0, 2, 1)
    box = _conv1d(r, params['box_w'], params['box_b']).transpose(0, 2, 1)

    scores = prob.reshape(-1)
    deltas = box.reshape(-1, 2)
    shifts = jnp.arange(L, dtype=jnp.float32)
    anc = (shifts[:, None, None] + BASE_ANCHORS[None, :, :]).reshape(-1, 2)
    w = anc[:, 1] - anc[:, 0] + 1.0
    ctr = anc[:, 0] + 0.5 * w
    pred_ctr = deltas[:, 0] * w + ctr
    pred_w = jnp.exp(jnp.clip(deltas[:, 1], -10.0, 10.0)) * w
    s = jnp.clip(pred_ctr - 0.5 * pred_w, 0.0, L - 1.0)
    e = jnp.clip(pred_ctr + 0.5 * pred_w, 0.0, L - 1.0)
    return scores, s, e


_NBLK = NSC // 128 // 128


def _thresh_body(sc_ref, out_ref):
    def count_ge(b):
        def blk(k, acc):
            x = sc_ref[pl.ds(k * 128, 128), :]
            xb = jax.lax.bitcast_convert_type(x, jnp.int32)
            return acc + (xb >= b).astype(jnp.int32)
        acc = lax.fori_loop(0, _NBLK, blk, jnp.zeros((128, 128), jnp.int32))
        return jnp.sum(acc)

    def bs(it, lohi):
        lo, hi = lohi
        mid = (lo + hi) // 2
        big = count_ge(mid) >= PRE_N
        return jnp.where(big, mid, lo), jnp.where(big, hi, mid)

    lo, hi = lax.fori_loop(0, 31, bs, (jnp.int32(0), jnp.int32(0x3F800001)))
    tb = lo
    b6 = PRE_N - count_ge(tb + 1)

    rowi = jax.lax.broadcasted_iota(jnp.int32, (128, 128), 0)
    lanei = jax.lax.broadcasted_iota(jnp.int32, (128, 128), 1)

    def count_eq_le(K):
        def blk(k, acc):
            x = sc_ref[pl.ds(k * 128, 128), :]
            xb = jax.lax.bitcast_convert_type(x, jnp.int32)
            gidx = (k * 128 + rowi) * 128 + lanei
            return acc + ((xb == tb) & (gidx <= K)).astype(jnp.int32)
        acc = lax.fori_loop(0, _NBLK, blk, jnp.zeros((128, 128), jnp.int32))
        return jnp.sum(acc)

    def bs2(it, lohi):
        lo, hi = lohi
        mid = (lo + hi) // 2
        ok = count_eq_le(mid) >= b6
        return jnp.where(ok, lo, mid), jnp.where(ok, hi, mid)

    lo2, hi2 = lax.fori_loop(0, 20, bs2, (jnp.int32(-1), jnp.int32(NSC - 1)))
    lane = jax.lax.broadcasted_iota(jnp.int32, (1, 128), 1)
    out_ref[...] = jnp.where(lane == 0, tb, jnp.where(lane == 1, hi2, 0))


def _threshold(scores):
    bk = pl.pallas_call(
        _thresh_body,
        out_shape=jax.ShapeDtypeStruct((1, 128), jnp.int32),
    )(scores.reshape(NSC // 128, 128))
    t = jax.lax.bitcast_convert_type(bk[0, 0], jnp.float32)
    return t, bk[0, 1]


def _compact_sc_body(score_hbm, t_hbm, k_hbm, outidx_hbm, counts_hbm, sc_vm, idx_vm, t_vm, k_vm, cnt_vm):
    wid = lax.axis_index("s") * 2 + lax.axis_index("c")
    base = wid * CHUNK
    pltpu.sync_copy(score_hbm.at[pl.ds(base, CHUNK)], sc_vm)
    pltpu.sync_copy(t_hbm, t_vm)
    pltpu.sync_copy(k_hbm, k_vm)
    t = t_vm[...]
    kv = k_vm[...]
    iota16 = lax.iota(jnp.int32, 16)

    def step(j, cursor):
        lidx = j * 16 + iota16
        v = plsc.load_gather(sc_vm, [lidx])
        gidx = base + lidx
        m = (v > t) | ((v == t) & (gidx <= kv))
        pos = cursor + plsc.cumsum(m.astype(jnp.int32)) - 1
        plsc.store_scatter(idx_vm, [pos], gidx, mask=m)
        return cursor + jnp.sum(m.astype(jnp.int32))

    cursor = lax.fori_loop(0, TROWS, step, jnp.int32(0))
    cnt_vm[...] = jnp.full((16,), cursor, jnp.int32)
    pltpu.sync_copy(cnt_vm, counts_hbm.at[wid])
    pltpu.sync_copy(idx_vm, outidx_hbm.at[wid])


def _compact(scores, t, K):
    t16 = jnp.full((16,), t, jnp.float32)
    k16 = jnp.full((16,), K, jnp.int32)
    mesh = plsc.VectorSubcoreMesh(core_axis_name="c", subcore_axis_name="s")
    kfn = pl.kernel(
        _compact_sc_body,
        out_type=[
            jax.ShapeDtypeStruct((NTILES, NPAD), jnp.int32),
            jax.ShapeDtypeStruct((NTILES, 16), jnp.int32),
        ],
        mesh=mesh,
        compiler_params=pltpu.CompilerParams(needs_layout_passes=False),
        scratch_types=[
            pltpu.VMEM((CHUNK,), jnp.float32),
            pltpu.VMEM((NPAD,), jnp.int32),
            pltpu.VMEM((16,), jnp.float32),
            pltpu.VMEM((16,), jnp.int32),
            pltpu.VMEM((16,), jnp.int32),
        ],
    )
    outidx, counts = kfn(scores, t16, k16)
    return outidx, counts[:, 0]


def _nms_sel_body(scr, ssr, eer, scc, ssc, eec, osc, oss, oee, live_ref):
    osc[...] = jnp.zeros_like(osc)
    oss[...] = jnp.zeros_like(oss)
    oee[...] = jnp.zeros_like(oee)
    live_ref[...] = scr[...]
    s_row = ssr[0:1, :]
    e_row = eer[0:1, :]
    lens = e_row - s_row + 1.0
    lane = jax.lax.broadcasted_iota(jnp.int32, (1, NPAD), 1)

    def cond_fn(st):
        cursor, done = st
        return jnp.logical_and(cursor < POST_N, done == 0)

    def body_fn(st):
        cursor, done = st
        live = live_ref[0:1, :]
        mx = jnp.max(live)
        is_done = mx < -0.5

        @pl.when(jnp.logical_not(is_done))
        def _():
            eq = live == mx
            p = jnp.min(jnp.where(eq, lane, jnp.int32(NPAD)))
            si = ssc[pl.ds(p, 1), :]
            ei = eec[pl.ds(p, 1), :]
            li = ei - si + 1.0
            inter = jnp.maximum(0.0, jnp.minimum(ei, e_row) - jnp.maximum(si, s_row) + 1.0)
            iou = inter / (li + lens - inter)
            live_ref[0:1, :] = jnp.where(iou > NMS_THRESH, -1.0, live)
            osc[pl.ds(cursor, 1), :] = scc[pl.ds(p, 1), :]
            oss[pl.ds(cursor, 1), :] = si
            oee[pl.ds(cursor, 1), :] = ei

        return cursor + jnp.where(is_done, 0, 1).astype(jnp.int32), jnp.where(is_done, 1, 0).astype(jnp.int32)

    lax.while_loop(cond_fn, body_fn, (jnp.int32(0), jnp.int32(0)))


def _nms_topk(sc, ss, ee):
    pad = NPAD - PRE_N
    neg = jnp.full((pad,), -3.0e9, jnp.float32)
    scr = jnp.concatenate([sc, jnp.full((pad,), -1.0, jnp.float32)]).reshape(1, NPAD)
    ssr = jnp.concatenate([ss, neg]).reshape(1, NPAD)
    eer = jnp.concatenate([ee, neg]).reshape(1, NPAD)
    scc = scr.reshape(NPAD, 1)
    ssc = ssr.reshape(NPAD, 1)
    eec = eer.reshape(NPAD, 1)
    out = pl.pallas_call(
        _nms_sel_body,
        out_shape=[jax.ShapeDtypeStruct((OUT_R, 1), jnp.float32)] * 3,
        scratch_shapes=[pltpu.VMEM((1, NPAD), jnp.float32)],
    )(scr, ssr, eer, scc, ssc, eec)
    osc, oss, oee = out
    return jnp.stack([osc[:POST_N, 0], oss[:POST_N, 0], oee[:POST_N, 0]], axis=-1)


def kernel(sequence, params):
    scores, s, e = _backbone(sequence, params)
    t, K = _threshold(scores)
    outidx, cnt = _compact(scores, t, K)
    offs0 = jnp.concatenate([jnp.zeros((1,), jnp.int32), jnp.cumsum(cnt)])
    p = jnp.arange(PRE_N, dtype=jnp.int32)
    tile = jnp.searchsorted(offs0, p, side='right').astype(jnp.int32) - 1
    src = tile * NPAD + (p - offs0[tile])
    fidx = outidx.reshape(-1)[src]
    out3 = _nms_topk(scores[fidx], s[fidx], e[fidx])
    return out3[:, None, :]
